# Initial kernel scaffold; baseline (speedup 1.0000x reference)
import jax
import jax.numpy as jnp
from jax.experimental import pallas as pl


def kernel(matched_word_ids, word_embedding_weight):
    raise NotImplementedError("write your pallas kernel here")


# speedup vs baseline: 4.1382x; 4.1382x over previous
import functools

import jax
import jax.numpy as jnp
from jax import lax
from jax.experimental import pallas as pl
from jax.experimental.pallas import tpu as pltpu
from jax.experimental.pallas import tpu_sc as plsc

D = 64
B = 4096 * 200
NC, NS = 2, 16
NW = NC * NS
B_PER_W = B // NW
CH = 1024
NCH = B_PER_W // CH

_mesh = plsc.VectorSubcoreMesh(core_axis_name="c", subcore_axis_name="s")


@functools.partial(
    pl.kernel,
    mesh=_mesh,
    out_type=jax.ShapeDtypeStruct((B, D), jnp.float32),
    scratch_types=[
        pltpu.VMEM((CH,), jnp.int32),
        pltpu.VMEM((CH, D), jnp.float32),
        pltpu.SemaphoreType.DMA,
    ],
    compiler_params=pltpu.CompilerParams(use_tc_tiling_on_sc=False),
)
def _gather_kernel(idx_hbm, table_hbm, out_hbm, idx_v, rows_v, sem):
    wid = lax.axis_index("s") * NC + lax.axis_index("c")
    base = wid * B_PER_W

    def body(i, carry):
        off = base + i * CH
        pltpu.sync_copy(idx_hbm.at[pl.ds(off, CH)], idx_v)
        pltpu.async_copy(table_hbm.at[idx_v], rows_v, sem).wait()
        pltpu.sync_copy(rows_v, out_hbm.at[pl.ds(off, CH)])
        return carry

    lax.fori_loop(0, NCH, body, 0)


def kernel(matched_word_ids, word_embedding_weight):
    idx = matched_word_ids.reshape(-1).astype(jnp.int32)
    out = _gather_kernel(idx, word_embedding_weight)
    return out.reshape(matched_word_ids.shape + (word_embedding_weight.shape[1],))


# speedup vs baseline: 4.2334x; 1.0230x over previous
import functools

import jax
import jax.numpy as jnp
from jax import lax
from jax.experimental import pallas as pl
from jax.experimental.pallas import tpu as pltpu
from jax.experimental.pallas import tpu_sc as plsc

D = 64
B = 4096 * 200
NC, NS = 2, 16
NW = NC * NS
B_PER_W = B // NW
CH = 320
NB = 4
NCH = B_PER_W // CH
assert B_PER_W % CH == 0 and NCH % NB == 0

_mesh = plsc.VectorSubcoreMesh(core_axis_name="c", subcore_axis_name="s")


@functools.partial(
    pl.kernel,
    mesh=_mesh,
    out_type=jax.ShapeDtypeStruct((B, D), jnp.float32),
    scratch_types=[
        pltpu.VMEM((B_PER_W,), jnp.int32),
        pltpu.VMEM((CH, D), jnp.float32),
        pltpu.VMEM((CH, D), jnp.float32),
        pltpu.VMEM((CH, D), jnp.float32),
        pltpu.VMEM((CH, D), jnp.float32),
        pltpu.SemaphoreType.DMA,
        pltpu.SemaphoreType.DMA,
        pltpu.SemaphoreType.DMA,
        pltpu.SemaphoreType.DMA,
        pltpu.SemaphoreType.DMA,
        pltpu.SemaphoreType.DMA,
        pltpu.SemaphoreType.DMA,
        pltpu.SemaphoreType.DMA,
    ],
    compiler_params=pltpu.CompilerParams(use_tc_tiling_on_sc=False),
)
def _gather_kernel(idx_hbm, table_hbm, out_hbm, idx_all,
                   r0, r1, r2, r3, sg0, sg1, sg2, sg3, sw0, sw1, sw2, sw3):
    rows = (r0, r1, r2, r3)
    sg = (sg0, sg1, sg2, sg3)
    sw = (sw0, sw1, sw2, sw3)

    wid = lax.axis_index("s") * NC + lax.axis_index("c")
    base = wid * B_PER_W

    pltpu.sync_copy(idx_hbm.at[pl.ds(base, B_PER_W)], idx_all)

    def gather_desc(c, b):
        return pltpu.make_async_copy(
            table_hbm.at[idx_all.at[pl.ds(c * CH, CH)]], rows[b], sg[b])

    def write_desc(c, b):
        return pltpu.make_async_copy(
            rows[b], out_hbm.at[pl.ds(base + c * CH, CH)], sw[b])

    for b in range(NB):
        gather_desc(b, b).start()

    def body(it, carry):
        g = it * NB
        for b in range(NB):
            c = g + b
            gather_desc(c, b).wait()
            write_desc(c, b).start()
        for b in range(NB):
            c = g + b
            write_desc(c, b).wait()

            @pl.when(c + NB < NCH)
            def _():
                gather_desc(c + NB, b).start()

        return carry

    lax.fori_loop(0, NCH // NB, body, 0)


def kernel(matched_word_ids, word_embedding_weight):
    idx = matched_word_ids.reshape(-1).astype(jnp.int32)
    out = _gather_kernel(idx, word_embedding_weight)
    return out.reshape(matched_word_ids.shape + (word_embedding_weight.shape[1],))


# speedup vs baseline: 4.2461x; 1.0030x over previous
import functools

import jax
import jax.numpy as jnp
from jax import lax
from jax.experimental import pallas as pl
from jax.experimental.pallas import tpu as pltpu
from jax.experimental.pallas import tpu_sc as plsc

D = 64
BATCH = 4096
HIST = 200
B = BATCH * HIST
NC, NS = 2, 16
NW = NC * NS
B_PER_W = B // NW
NBA = BATCH // NW
CH = HIST
NB = 4
NCH = NBA
assert NCH % NB == 0

_mesh = plsc.VectorSubcoreMesh(core_axis_name="c", subcore_axis_name="s")


@functools.partial(
    pl.kernel,
    mesh=_mesh,
    out_type=jax.ShapeDtypeStruct((BATCH, HIST, D), jnp.float32),
    scratch_types=[
        pltpu.VMEM((B_PER_W,), jnp.int32),
        pltpu.VMEM((CH, D), jnp.float32),
        pltpu.VMEM((CH, D), jnp.float32),
        pltpu.VMEM((CH, D), jnp.float32),
        pltpu.VMEM((CH, D), jnp.float32),
        pltpu.SemaphoreType.DMA,
        pltpu.SemaphoreType.DMA,
        pltpu.SemaphoreType.DMA,
        pltpu.SemaphoreType.DMA,
        pltpu.SemaphoreType.DMA,
        pltpu.SemaphoreType.DMA,
        pltpu.SemaphoreType.DMA,
        pltpu.SemaphoreType.DMA,
    ],
    compiler_params=pltpu.CompilerParams(use_tc_tiling_on_sc=False),
)
def _gather_kernel(idx_hbm, table_hbm, out_hbm, idx_all,
                   r0, r1, r2, r3, sg0, sg1, sg2, sg3, sw0, sw1, sw2, sw3):
    rows = (r0, r1, r2, r3)
    sg = (sg0, sg1, sg2, sg3)
    sw = (sw0, sw1, sw2, sw3)

    wid = lax.axis_index("s") * NC + lax.axis_index("c")
    base = wid * B_PER_W
    batch_base = wid * NBA

    pltpu.sync_copy(idx_hbm.at[pl.ds(base, B_PER_W)], idx_all)

    def gather_desc(c, b):
        return pltpu.make_async_copy(
            table_hbm.at[idx_all.at[pl.ds(c * CH, CH)]], rows[b], sg[b])

    def write_desc(c, b):
        return pltpu.make_async_copy(
            rows[b], out_hbm.at[batch_base + c], sw[b])

    for b in range(NB):
        gather_desc(b, b).start()

    def body(it, carry):
        g = it * NB
        for b in range(NB):
            c = g + b
            gather_desc(c, b).wait()
            write_desc(c, b).start()
        for b in range(NB):
            c = g + b
            write_desc(c, b).wait()

            @pl.when(c + NB < NCH)
            def _():
                gather_desc(c + NB, b).start()

        return carry

    lax.fori_loop(0, NCH // NB, body, 0)


def kernel(matched_word_ids, word_embedding_weight):
    idx = matched_word_ids.reshape(-1).astype(jnp.int32)
    return _gather_kernel(idx, word_embedding_weight)


# speedup vs baseline: 4.4734x; 1.0535x over previous
import functools

import jax
import jax.numpy as jnp
from jax import lax
from jax.experimental import pallas as pl
from jax.experimental.pallas import tpu as pltpu
from jax.experimental.pallas import tpu_sc as plsc

D = 64
BATCH = 4096
HIST = 200
B = BATCH * HIST
NC, NS = 2, 16
NW = NC * NS
B_PER_W = B // NW
CH = 320
NB = 4
NCH = B_PER_W // CH
assert B_PER_W % CH == 0 and NCH % NB == 0 and CH % 8 == 0

_mesh = plsc.VectorSubcoreMesh(core_axis_name="c", subcore_axis_name="s")


@functools.partial(
    pl.kernel,
    mesh=_mesh,
    out_type=jax.ShapeDtypeStruct((B, D), jnp.float32),
    scratch_types=[
        pltpu.VMEM((B_PER_W,), jnp.int32),
        pltpu.VMEM((CH, D), jnp.float32),
        pltpu.VMEM((CH, D), jnp.float32),
        pltpu.VMEM((CH, D), jnp.float32),
        pltpu.VMEM((CH, D), jnp.float32),
        pltpu.SemaphoreType.DMA,
        pltpu.SemaphoreType.DMA,
        pltpu.SemaphoreType.DMA,
        pltpu.SemaphoreType.DMA,
        pltpu.SemaphoreType.DMA,
        pltpu.SemaphoreType.DMA,
        pltpu.SemaphoreType.DMA,
        pltpu.SemaphoreType.DMA,
    ],
    compiler_params=pltpu.CompilerParams(use_tc_tiling_on_sc=False),
)
def _sc_gather(idx_hbm, table_hbm, out_hbm, idx_all,
               r0, r1, r2, r3, sg0, sg1, sg2, sg3, sw0, sw1, sw2, sw3):
    rows = (r0, r1, r2, r3)
    sg = (sg0, sg1, sg2, sg3)
    sw = (sw0, sw1, sw2, sw3)

    wid = lax.axis_index("s") * NC + lax.axis_index("c")
    base = wid * B_PER_W

    pltpu.sync_copy(idx_hbm.at[pl.ds(base, B_PER_W)], idx_all)

    def gather_desc(c, b):
        return pltpu.make_async_copy(
            table_hbm.at[idx_all.at[pl.ds(c * CH, CH)]], rows[b], sg[b])

    def write_desc(c, b):
        return pltpu.make_async_copy(
            rows[b], out_hbm.at[pl.ds(base + c * CH, CH)], sw[b])

    for b in range(NB):
        gather_desc(b, b).start()

    def body(it, carry):
        g = it * NB
        for b in range(NB):
            c = g + b
            gather_desc(c, b).wait()
            write_desc(c, b).start()
        for b in range(NB):
            c = g + b
            write_desc(c, b).wait()

            @pl.when(c + NB < NCH)
            def _():
                gather_desc(c + NB, b).start()

        return carry

    lax.fori_loop(0, NCH // NB, body, 0)


_HB = BATCH // 2


def _tc_transpose_body(g_ref, o_ref):
    xt = g_ref[...].T
    o_ref[0, :, 0:_HB] = xt[0:D, :]
    o_ref[0, :, _HB:BATCH] = xt[D:2 * D, :]


_tc_transpose = pl.pallas_call(
    _tc_transpose_body,
    grid=(HIST,),
    in_specs=[pl.BlockSpec((_HB, 2 * D), lambda h: (h, 0))],
    out_specs=pl.BlockSpec((1, D, BATCH), lambda h: (h, 0, 0)),
    out_shape=jax.ShapeDtypeStruct((HIST, D, BATCH), jnp.float32),
)


def kernel(matched_word_ids, word_embedding_weight):
    idx_t = (matched_word_ids.T.reshape(HIST, 2, _HB)
             .transpose(0, 2, 1).reshape(-1).astype(jnp.int32))
    g = _sc_gather(idx_t, word_embedding_weight)
    g2 = g.reshape(B // 2, 2 * D)
    out_t = _tc_transpose(g2)
    return jnp.transpose(out_t, (2, 0, 1))


# speedup vs baseline: 5.1565x; 1.1527x over previous
import functools

import jax
import jax.numpy as jnp
from jax import lax
from jax.experimental import pallas as pl
from jax.experimental.pallas import tpu as pltpu
from jax.experimental.pallas import tpu_sc as plsc

D = 64
BATCH = 4096
HIST = 200
B = BATCH * HIST
NC, NS = 2, 16
NW = NC * NS
B_PER_W = B // NW
CH = 320
NB = 4
NCH = B_PER_W // CH
assert B_PER_W % CH == 0 and NCH % NB == 0 and CH % 8 == 0

_mesh = plsc.VectorSubcoreMesh(core_axis_name="c", subcore_axis_name="s")


@functools.partial(
    pl.kernel,
    mesh=_mesh,
    out_type=jax.ShapeDtypeStruct((B, D), jnp.float32),
    scratch_types=[
        pltpu.VMEM((B_PER_W,), jnp.int32),
        pltpu.VMEM((CH, D), jnp.float32),
        pltpu.VMEM((CH, D), jnp.float32),
        pltpu.VMEM((CH, D), jnp.float32),
        pltpu.VMEM((CH, D), jnp.float32),
        pltpu.SemaphoreType.DMA,
        pltpu.SemaphoreType.DMA,
        pltpu.SemaphoreType.DMA,
        pltpu.SemaphoreType.DMA,
        pltpu.SemaphoreType.DMA,
        pltpu.SemaphoreType.DMA,
        pltpu.SemaphoreType.DMA,
        pltpu.SemaphoreType.DMA,
    ],
    compiler_params=pltpu.CompilerParams(use_tc_tiling_on_sc=False),
)
def _sc_gather(idx_hbm, table_hbm, out_hbm, idx_all,
               r0, r1, r2, r3, sg0, sg1, sg2, sg3, sw0, sw1, sw2, sw3):
    rows = (r0, r1, r2, r3)
    sg = (sg0, sg1, sg2, sg3)
    sw = (sw0, sw1, sw2, sw3)

    wid = lax.axis_index("s") * NC + lax.axis_index("c")
    base = wid * B_PER_W

    pltpu.sync_copy(idx_hbm.at[pl.ds(base, B_PER_W)], idx_all)

    def gather_desc(c, b):
        return pltpu.make_async_copy(
            table_hbm.at[idx_all.at[pl.ds(c * CH, CH)]], rows[b], sg[b])

    def write_desc(c, b):
        return pltpu.make_async_copy(
            rows[b], out_hbm.at[pl.ds(base + c * CH, CH)], sw[b])

    for b in range(NB):
        gather_desc(b, b).start()

    def body(it, carry):
        g = it * NB
        for b in range(NB):
            c = g + b
            gather_desc(c, b).wait()
            write_desc(c, b).start()
        for b in range(NB):
            c = g + b
            write_desc(c, b).wait()

            @pl.when(c + NB < NCH)
            def _():
                gather_desc(c + NB, b).start()

        return carry

    lax.fori_loop(0, NCH // NB, body, 0)


def _tc_transpose_body(g_ref, o_ref):
    xt = g_ref[...].T
    o_ref[0] = xt[0:D, :]
    o_ref[1] = xt[D:2 * D, :]


_tc_transpose = pl.pallas_call(
    _tc_transpose_body,
    grid=(HIST // 2,),
    in_specs=[pl.BlockSpec((BATCH, 2 * D), lambda k: (0, k))],
    out_specs=pl.BlockSpec((2, D, BATCH), lambda k: (k, 0, 0)),
    out_shape=jax.ShapeDtypeStruct((HIST, D, BATCH), jnp.float32),
)


def kernel(matched_word_ids, word_embedding_weight):
    idx = matched_word_ids.reshape(-1).astype(jnp.int32)
    g = _sc_gather(idx, word_embedding_weight)
    g2 = g.reshape(BATCH, HIST * D)
    out_t = _tc_transpose(g2)
    return jnp.transpose(out_t, (2, 0, 1))


# speedup vs baseline: 7.8349x; 1.5194x over previous
import functools

import jax
import jax.numpy as jnp
from jax import lax
from jax.experimental import pallas as pl
from jax.experimental.pallas import tpu as pltpu
from jax.experimental.pallas import tpu_sc as plsc

D = 64
BATCH = 4096
HIST = 200
HB = BATCH // 2
B = BATCH * HIST
NC, NS = 2, 16
NW = NC * NS
B_PER_W = B // NW
CH = 256
NB = 4
NCH = B_PER_W // CH
assert B_PER_W % CH == 0 and NCH % NB == 0 and HB % CH == 0

_mesh = plsc.VectorSubcoreMesh(core_axis_name="c", subcore_axis_name="s")


@functools.partial(
    pl.kernel,
    mesh=_mesh,
    out_type=jax.ShapeDtypeStruct((B // 2, 2 * D), jnp.float32),
    scratch_types=[
        pltpu.VMEM((B_PER_W,), jnp.int32),
        pltpu.VMEM((CH, D), jnp.float32),
        pltpu.VMEM((CH, D), jnp.float32),
        pltpu.VMEM((CH, D), jnp.float32),
        pltpu.VMEM((CH, D), jnp.float32),
        pltpu.SemaphoreType.DMA,
        pltpu.SemaphoreType.DMA,
        pltpu.SemaphoreType.DMA,
        pltpu.SemaphoreType.DMA,
        pltpu.SemaphoreType.DMA,
        pltpu.SemaphoreType.DMA,
        pltpu.SemaphoreType.DMA,
        pltpu.SemaphoreType.DMA,
    ],
    compiler_params=pltpu.CompilerParams(use_tc_tiling_on_sc=False),
)
def _sc_gather(idx_hbm, table_hbm, out_hbm, idx_all,
               r0, r1, r2, r3, sg0, sg1, sg2, sg3, sw0, sw1, sw2, sw3):
    rows = (r0, r1, r2, r3)
    sg = (sg0, sg1, sg2, sg3)
    sw = (sw0, sw1, sw2, sw3)

    wid = lax.axis_index("s") * NC + lax.axis_index("c")
    base = wid * B_PER_W

    pltpu.sync_copy(idx_hbm.at[pl.ds(base, B_PER_W)], idx_all)

    def gather_desc(c, b):
        return pltpu.make_async_copy(
            table_hbm.at[idx_all.at[pl.ds(c * CH, CH)]], rows[b], sg[b])

    def write_desc(c, b):
        q0 = base + c * CH
        h = q0 // BATCH
        p0 = q0 % BATCH
        s = p0 // HB
        r0 = p0 % HB
        drow = pl.multiple_of(h * HB + r0, CH)
        dcol = pl.multiple_of(s * D, D)
        return pltpu.make_async_copy(
            rows[b], out_hbm.at[pl.ds(drow, CH), pl.ds(dcol, D)], sw[b])

    for b in range(NB):
        gather_desc(b, b).start()

    def body(it, carry):
        g = it * NB
        for b in range(NB):
            c = g + b
            gather_desc(c, b).wait()
            write_desc(c, b).start()
        for b in range(NB):
            c = g + b
            write_desc(c, b).wait()

            @pl.when(c + NB < NCH)
            def _():
                gather_desc(c + NB, b).start()

        return carry

    lax.fori_loop(0, NCH // NB, body, 0)


def _tc_transpose_body(g_ref, o_ref):
    xt = g_ref[...].T
    o_ref[0, :, 0:HB] = xt[0:D, 0:HB]
    o_ref[0, :, HB:BATCH] = xt[D:2 * D, 0:HB]
    o_ref[1, :, 0:HB] = xt[0:D, HB:BATCH]
    o_ref[1, :, HB:BATCH] = xt[D:2 * D, HB:BATCH]


_tc_transpose = pl.pallas_call(
    _tc_transpose_body,
    grid=(HIST // 2,),
    in_specs=[pl.BlockSpec((BATCH, 2 * D), lambda k: (k, 0))],
    out_specs=pl.BlockSpec((2, D, BATCH), lambda k: (k, 0, 0)),
    out_shape=jax.ShapeDtypeStruct((HIST, D, BATCH), jnp.float32),
)


def kernel(matched_word_ids, word_embedding_weight):
    idx_t = matched_word_ids.T.reshape(-1).astype(jnp.int32)
    g2 = _sc_gather(idx_t, word_embedding_weight)
    out_t = _tc_transpose(g2)
    return jnp.transpose(out_t, (2, 0, 1))


# speedup vs baseline: 7.9453x; 1.0141x over previous
import functools

import jax
import jax.numpy as jnp
from jax import lax
from jax.experimental import pallas as pl
from jax.experimental.pallas import tpu as pltpu
from jax.experimental.pallas import tpu_sc as plsc

D = 64
BATCH = 4096
HIST = 200
HB = BATCH // 2
B = BATCH * HIST
NSPLIT = 2
HSPLIT = HIST // NSPLIT
BS = B // NSPLIT
NC, NS = 2, 16
NW = NC * NS
B_PER_W = BS // NW
CH = 256
NB = 2
NCH = B_PER_W // CH
assert B_PER_W % CH == 0 and NCH % NB == 0 and HB % CH == 0

_mesh = plsc.VectorSubcoreMesh(core_axis_name="c", subcore_axis_name="s")


def _make_sc_gather(h0):
    @functools.partial(
        pl.kernel,
        mesh=_mesh,
        out_type=jax.ShapeDtypeStruct((BS // 2, 2 * D), jnp.float32),
        scratch_types=[
            pltpu.VMEM((B_PER_W,), jnp.int32),
            pltpu.VMEM((CH, D), jnp.float32),
            pltpu.VMEM((CH, D), jnp.float32),
            pltpu.SemaphoreType.DMA,
            pltpu.SemaphoreType.DMA,
            pltpu.SemaphoreType.DMA,
            pltpu.SemaphoreType.DMA,
        ],
        compiler_params=pltpu.CompilerParams(use_tc_tiling_on_sc=False),
    )
    def _sc_gather(idx_hbm, table_hbm, out_hbm, idx_all,
                   r0, r1, sg0, sg1, sw0, sw1):
        rows = (r0, r1)
        sg = (sg0, sg1)
        sw = (sw0, sw1)

        wid = lax.axis_index("s") * NC + lax.axis_index("c")
        base = h0 * BATCH + wid * B_PER_W

        pltpu.sync_copy(idx_hbm.at[pl.ds(base, B_PER_W)], idx_all)

        def gather_desc(c, b):
            return pltpu.make_async_copy(
                table_hbm.at[idx_all.at[pl.ds(c * CH, CH)]], rows[b], sg[b])

        def write_desc(c, b):
            q0 = base + c * CH
            h = q0 // BATCH - h0
            p0 = q0 % BATCH
            s = p0 // HB
            r0_ = p0 % HB
            drow = pl.multiple_of(h * HB + r0_, CH)
            dcol = pl.multiple_of(s * D, D)
            return pltpu.make_async_copy(
                rows[b], out_hbm.at[pl.ds(drow, CH), pl.ds(dcol, D)], sw[b])

        for b in range(NB):
            gather_desc(b, b).start()

        def body(it, carry):
            g = it * NB
            for b in range(NB):
                c = g + b
                gather_desc(c, b).wait()
                write_desc(c, b).start()
            for b in range(NB):
                c = g + b
                write_desc(c, b).wait()

                @pl.when(c + NB < NCH)
                def _():
                    gather_desc(c + NB, b).start()

            return carry

        lax.fori_loop(0, NCH // NB, body, 0)

    return _sc_gather


_sc_gather_halves = tuple(_make_sc_gather(i * HSPLIT) for i in range(NSPLIT))


def _tc_transpose_body_first(g_ref, o_ref):
    xt = g_ref[...].T
    o_ref[0, :, 0:HB] = xt[0:D, 0:HB]
    o_ref[0, :, HB:BATCH] = xt[D:2 * D, 0:HB]
    o_ref[1, :, 0:HB] = xt[0:D, HB:BATCH]
    o_ref[1, :, HB:BATCH] = xt[D:2 * D, HB:BATCH]


def _tc_transpose_body_rest(g_ref, _prev_ref, o_ref):
    _tc_transpose_body_first(g_ref, o_ref)


def _make_tc_transpose(split, first):
    kb = split * (HSPLIT // 2)
    if first:
        return pl.pallas_call(
            _tc_transpose_body_first,
            grid=(HSPLIT // 2,),
            in_specs=[pl.BlockSpec((BATCH, 2 * D), lambda k: (k, 0))],
            out_specs=pl.BlockSpec((2, D, BATCH), lambda k: (k + kb, 0, 0)),
            out_shape=jax.ShapeDtypeStruct((HIST, D, BATCH), jnp.float32),
        )
    return pl.pallas_call(
        _tc_transpose_body_rest,
        grid=(HSPLIT // 2,),
        in_specs=[
            pl.BlockSpec((BATCH, 2 * D), lambda k: (k, 0)),
            pl.BlockSpec(memory_space=pl.ANY),
        ],
        out_specs=pl.BlockSpec((2, D, BATCH), lambda k: (k + kb, 0, 0)),
        out_shape=jax.ShapeDtypeStruct((HIST, D, BATCH), jnp.float32),
        input_output_aliases={1: 0},
    )


_tc_first = _make_tc_transpose(0, True)
_tc_rest = tuple(_make_tc_transpose(i, False) for i in range(1, NSPLIT))


def kernel(matched_word_ids, word_embedding_weight):
    idx_t = matched_word_ids.T.reshape(-1).astype(jnp.int32)
    g2s = [f(idx_t, word_embedding_weight) for f in _sc_gather_halves]
    out_t = _tc_first(g2s[0])
    for i in range(1, NSPLIT):
        out_t = _tc_rest[i - 1](g2s[i], out_t)
    return jnp.transpose(out_t, (2, 0, 1))


# speedup vs baseline: 8.1371x; 1.0241x over previous
import functools

import jax
import jax.numpy as jnp
from jax import lax
from jax.experimental import pallas as pl
from jax.experimental.pallas import tpu as pltpu
from jax.experimental.pallas import tpu_sc as plsc

D = 64
BATCH = 4096
HIST = 200
HB = BATCH // 2
B = BATCH * HIST
NSPLIT = 4
HSPLIT = HIST // NSPLIT
BS = B // NSPLIT
NC, NS = 2, 16
NW = NC * NS
B_PER_W = BS // NW
CH = 256
NB = 5
NCH = B_PER_W // CH
assert B_PER_W % CH == 0 and NCH % NB == 0 and HB % CH == 0

_mesh = plsc.VectorSubcoreMesh(core_axis_name="c", subcore_axis_name="s")


def _make_sc_gather(h0):
    @functools.partial(
        pl.kernel,
        mesh=_mesh,
        out_type=jax.ShapeDtypeStruct((BS // 2, 2 * D), jnp.float32),
        scratch_types=[
            pltpu.VMEM((B_PER_W,), jnp.int32),
        ]
        + [pltpu.VMEM((CH, D), jnp.float32)] * NB
        + [pltpu.SemaphoreType.DMA] * (2 * NB),
        compiler_params=pltpu.CompilerParams(use_tc_tiling_on_sc=False),
    )
    def _sc_gather(idx_hbm, table_hbm, out_hbm, idx_all, *bufs):
        rows = bufs[:NB]
        sg = bufs[NB:2 * NB]
        sw = bufs[2 * NB:3 * NB]

        wid = lax.axis_index("s") * NC + lax.axis_index("c")
        base = h0 * BATCH + wid * B_PER_W

        pltpu.sync_copy(idx_hbm.at[pl.ds(base, B_PER_W)], idx_all)

        def gather_desc(c, b):
            return pltpu.make_async_copy(
                table_hbm.at[idx_all.at[pl.ds(c * CH, CH)]], rows[b], sg[b])

        def write_desc(c, b):
            q0 = base + c * CH
            h = q0 // BATCH - h0
            p0 = q0 % BATCH
            s = p0 // HB
            r0_ = p0 % HB
            drow = pl.multiple_of(h * HB + r0_, CH)
            dcol = pl.multiple_of(s * D, D)
            return pltpu.make_async_copy(
                rows[b], out_hbm.at[pl.ds(drow, CH), pl.ds(dcol, D)], sw[b])

        for b in range(NB):
            gather_desc(b, b).start()

        def body(it, carry):
            g = it * NB
            for b in range(NB):
                c = g + b
                gather_desc(c, b).wait()
                write_desc(c, b).start()
            for b in range(NB):
                c = g + b
                write_desc(c, b).wait()

                @pl.when(c + NB < NCH)
                def _():
                    gather_desc(c + NB, b).start()

            return carry

        lax.fori_loop(0, NCH // NB, body, 0)

    return _sc_gather


_sc_gather_halves = tuple(_make_sc_gather(i * HSPLIT) for i in range(NSPLIT))


def _tc_transpose_body_first(g_ref, o_ref):
    xt = g_ref[...].T
    o_ref[0, :, 0:HB] = xt[0:D, 0:HB]
    o_ref[0, :, HB:BATCH] = xt[D:2 * D, 0:HB]
    o_ref[1, :, 0:HB] = xt[0:D, HB:BATCH]
    o_ref[1, :, HB:BATCH] = xt[D:2 * D, HB:BATCH]


def _tc_transpose_body_rest(g_ref, _prev_ref, o_ref):
    _tc_transpose_body_first(g_ref, o_ref)


def _make_tc_transpose(split, first):
    kb = split * (HSPLIT // 2)
    if first:
        return pl.pallas_call(
            _tc_transpose_body_first,
            grid=(HSPLIT // 2,),
            in_specs=[pl.BlockSpec((BATCH, 2 * D), lambda k: (k, 0))],
            out_specs=pl.BlockSpec((2, D, BATCH), lambda k: (k + kb, 0, 0)),
            out_shape=jax.ShapeDtypeStruct((HIST, D, BATCH), jnp.float32),
        )
    return pl.pallas_call(
        _tc_transpose_body_rest,
        grid=(HSPLIT // 2,),
        in_specs=[
            pl.BlockSpec((BATCH, 2 * D), lambda k: (k, 0)),
            pl.BlockSpec(memory_space=pl.ANY),
        ],
        out_specs=pl.BlockSpec((2, D, BATCH), lambda k: (k + kb, 0, 0)),
        out_shape=jax.ShapeDtypeStruct((HIST, D, BATCH), jnp.float32),
        input_output_aliases={1: 0},
    )


_tc_first = _make_tc_transpose(0, True)
_tc_rest = tuple(_make_tc_transpose(i, False) for i in range(1, NSPLIT))


def kernel(matched_word_ids, word_embedding_weight):
    idx_t = matched_word_ids.T.reshape(-1).astype(jnp.int32)
    g2s = [f(idx_t, word_embedding_weight) for f in _sc_gather_halves]
    out_t = _tc_first(g2s[0])
    for i in range(1, NSPLIT):
        out_t = _tc_rest[i - 1](g2s[i], out_t)
    return jnp.transpose(out_t, (2, 0, 1))
